# baseline (speedup 1.0000x reference)
import functools

import jax
import jax.numpy as jnp
import numpy as np
from jax import lax
from jax.experimental import pallas as pl
from jax.experimental.pallas import tpu as pltpu
from jax.experimental.pallas import tpu_sc as plsc

_NUM_OBJ = 4
_NUM_LEVELS = 16
_FPL = 2
_T = 1 << 19
_BASE_RES = 16
_GROWTH = 1.3819
_N = 524288

_PY = np.int32(np.uint32(2654435761).view(np.int32))
_PZ = np.int32(805459861)


def _levels():
    ress, sizes, offsets = [], [], []
    off = 0
    for l in range(_NUM_LEVELS):
        res = int(np.floor(_BASE_RES * (_GROWTH ** l)))
        nv = (res + 1) ** 3
        size = min(nv, _T)
        ress.append(res)
        sizes.append(size)
        offsets.append(off)
        off += size
    return ress, sizes, offsets, off


_RESS, _SIZES, _OFFSETS, _TOTAL_ROWS = _levels()
_N_DENSE = sum(1 for l in range(_NUM_LEVELS)
               if _SIZES[l] == (_RESS[l] + 1) ** 3)

_PIECE = 8192
_SWS = (_T // _PIECE + 1) * _PIECE
_AOFF = []
_cur = 0
for _l in range(_NUM_LEVELS):
    _AOFF.append(_cur)
    _cur = -(-(_cur + _SIZES[_l]) // 8) * 8
_DENSEW = _AOFF[_N_DENSE - 1] + _SIZES[_N_DENSE - 1]
_DPIECES = -(-_DENSEW // _PIECE)
_RPAD = _AOFF[_NUM_LEVELS - 1] + _SWS

_NC, _NS = 2, 16
_NW = _NC * _NS
_C = 256
_PPW = _N // _NW
_NCH = _PPW // _C
_HALF = _NCH // 2

_CORNERS = [(dx, dy, dz) for dx in (0, 1) for dy in (0, 1) for dz in (0, 1)]


def _sc_body(xs_hbm, ys_hbm, zs_hbm, tab_hbm, out_hbm,
             px, py, pz, idx_a, idx_b, rows_a, rows_b, wt_a, wt_b, out_h,
             bounce_a, bounce_b, spm, sem_a, sem_b):
    cid = lax.axis_index("c")
    sid = lax.axis_index("s")
    wid = cid * _NS + sid

    pstart = pl.multiple_of(wid * _PPW, 8)
    pltpu.sync_copy(xs_hbm.at[pl.ds(pstart, _PPW)], px)
    pltpu.sync_copy(ys_hbm.at[pl.ds(pstart, _PPW)], py)
    pltpu.sync_copy(zs_hbm.at[pl.ds(pstart, _PPW)], pz)

    def stage(start, npieces):
        bufs, sems = [bounce_a, bounce_b], [sem_a, sem_b]

        def fetch(k):
            p = k * _NS + sid

            @pl.when(p < npieces)
            def _():
                src = pl.multiple_of(start + p * _PIECE, 8)
                pltpu.async_copy(tab_hbm.at[pl.ds(src, _PIECE)],
                                 bufs[k % 2], sems[k % 2])

        def push(k):
            p = k * _NS + sid

            @pl.when(p < npieces)
            def _():
                pltpu.make_async_copy(tab_hbm.at[pl.ds(0, _PIECE)],
                                      bufs[k % 2], sems[k % 2]).wait()
                dst = pl.multiple_of(p * _PIECE, 8)
                pltpu.sync_copy(bufs[k % 2], spm.at[pl.ds(dst, _PIECE)])

        nk = -(-npieces // _NS)
        fetch(0)
        for k in range(1, nk):
            fetch(k)
            push(k - 1)
        push(nk - 1)

    stage(0, _DPIECES)
    plsc.subcore_barrier()

    def compute_idx(l, ck, idx_buf, wt_buf):
        res, size = _RESS[l], _SIZES[l]
        dense = size == (res + 1) ** 3
        rel = _AOFF[l] if dense else 0
        fr = jnp.float32(res)

        def g_body(g, c_):
            s = g * 16
            o = ck * _C + s
            x = px[pl.ds(o, 16)]
            y = py[pl.ds(o, 16)]
            z = pz[pl.ds(o, 16)]
            xf, yf, zf = x * fr, y * fr, z * fr
            xi = xf.astype(jnp.int32)
            yi = yf.astype(jnp.int32)
            zi = zf.astype(jnp.int32)
            wx = xf - xi.astype(jnp.float32)
            wy = yf - yi.astype(jnp.float32)
            wz = zf - zi.astype(jnp.float32)
            cwx, cwy, cwz = 1.0 - wx, 1.0 - wy, 1.0 - wz
            wyz = {(0, 0): cwy * cwz, (0, 1): cwy * wz,
                   (1, 0): wy * cwz, (1, 1): wy * wz}
            for c, (dx, dy, dz) in enumerate(_CORNERS):
                wt_buf[pl.ds(c * _C + s, 16)] = \
                    (wx if dx else cwx) * wyz[(dy, dz)]
            cx = jnp.minimum(xi, res - 1)
            cy = jnp.minimum(yi, res - 1)
            cz = jnp.minimum(zi, res - 1)
            if dense:
                r1 = res + 1
                a00 = cy + r1 * cz
                ra = {(0, 0): r1 * a00 + rel,
                      (0, 1): r1 * a00 + r1 * r1 + rel,
                      (1, 0): r1 * a00 + r1 + rel,
                      (1, 1): r1 * a00 + r1 * r1 + r1 + rel}
                cx1 = cx + 1
                for c, (dx, dy, dz) in enumerate(_CORNERS):
                    idx_buf[pl.ds(c * _C + s, 16)] = \
                        (cx1 if dx else cx) + ra[(dy, dz)]
            else:
                mask = size - 1
                hx0, hx1 = cx, cx + 1
                hy0 = cy * _PY
                hy1 = hy0 + _PY
                hz0 = cz * _PZ
                hz1 = hz0 + _PZ
                e = {(0, 0): hy0 ^ hz0, (0, 1): hy0 ^ hz1,
                     (1, 0): hy1 ^ hz0, (1, 1): hy1 ^ hz1}
                for c, (dx, dy, dz) in enumerate(_CORNERS):
                    h = (hx1 if dx else hx0) ^ e[(dy, dz)]
                    idx_buf[pl.ds(c * _C + s, 16)] = h & mask
            return c_

        lax.fori_loop(0, _C // 16, g_body, 0)

    def accumulate(l, ck, rows_buf, wt_buf):
        himask = jnp.int32(-65536)

        def g_body(g, c_):
            s = g * 16
            acc0 = jnp.zeros((16,), jnp.float32)
            acc1 = jnp.zeros((16,), jnp.float32)
            for c, (dx, dy, dz) in enumerate(_CORNERS):
                wt = wt_buf[pl.ds(c * _C + s, 16)]
                w = lax.bitcast_convert_type(rows_buf[pl.ds(c * _C + s, 16)],
                                             jnp.int32)
                f0 = lax.bitcast_convert_type(lax.shift_left(w, 16),
                                              jnp.float32)
                f1 = lax.bitcast_convert_type(w & himask, jnp.float32)
                acc0 = acc0 + wt * f0
                acc1 = acc1 + wt * f1
            oo = (ck & (_HALF - 1)) * _C + s
            out_h[0, pl.ds(oo, 16)] = acc0
            out_h[1, pl.ds(oo, 16)] = acc1
            return c_

        lax.fori_loop(0, _C // 16, g_body, 0)

    def flush(l, ck):
        @pl.when((ck & (_HALF - 1)) == _HALF - 1)
        def _():
            off = pl.multiple_of(wid * _PPW + (ck // _HALF) * (_HALF * _C), 8)
            pltpu.sync_copy(out_h.at[0],
                            out_hbm.at[2 * l].at[pl.ds(off, _HALF * _C)])
            pltpu.sync_copy(out_h.at[1],
                            out_hbm.at[2 * l + 1].at[pl.ds(off, _HALF * _C)])

    for l in range(_NUM_LEVELS):
        if l >= _N_DENSE:
            plsc.subcore_barrier()
            stage(_AOFF[l], _SWS // _PIECE)
            plsc.subcore_barrier()
        src = spm

        compute_idx(l, 0, idx_a, wt_a)
        pltpu.async_copy(src.at[idx_a], rows_a, sem_a)

        def pair_body(j, c_, l=l, src=src):
            ck_b = 2 * j + 1
            compute_idx(l, ck_b, idx_b, wt_b)
            pltpu.async_copy(src.at[idx_b], rows_b, sem_b)
            pltpu.make_async_copy(src.at[idx_a], rows_a, sem_a).wait()
            accumulate(l, 2 * j, rows_a, wt_a)
            ck_a = jnp.minimum(2 * j + 2, _NCH - 1)
            compute_idx(l, ck_a, idx_a, wt_a)
            pltpu.async_copy(src.at[idx_a], rows_a, sem_a)
            pltpu.make_async_copy(src.at[idx_b], rows_b, sem_b).wait()
            accumulate(l, ck_b, rows_b, wt_b)
            flush(l, ck_b)
            return c_

        lax.fori_loop(0, _NCH // 2, pair_body, 0)
        pltpu.make_async_copy(src.at[idx_a], rows_a, sem_a).wait()


_hashgrid_sc = functools.partial(
    pl.kernel,
    out_type=jax.ShapeDtypeStruct((2 * _NUM_LEVELS, _N), jnp.float32),
    mesh=plsc.VectorSubcoreMesh(core_axis_name="c", subcore_axis_name="s",
                                num_cores=_NC, num_subcores=_NS),
    scratch_types=[
        pltpu.VMEM((_PPW,), jnp.float32),
        pltpu.VMEM((_PPW,), jnp.float32),
        pltpu.VMEM((_PPW,), jnp.float32),
        pltpu.VMEM((8 * _C,), jnp.int32),
        pltpu.VMEM((8 * _C,), jnp.int32),
        pltpu.VMEM((8 * _C,), jnp.float32),
        pltpu.VMEM((8 * _C,), jnp.float32),
        pltpu.VMEM((8 * _C,), jnp.float32),
        pltpu.VMEM((8 * _C,), jnp.float32),
        pltpu.VMEM((2, _HALF * _C), jnp.float32),
        pltpu.VMEM((_PIECE,), jnp.float32),
        pltpu.VMEM((_PIECE,), jnp.float32),
        pltpu.VMEM_SHARED((_SWS,), jnp.float32),
        pltpu.SemaphoreType.DMA,
        pltpu.SemaphoreType.DMA,
    ],
)(_sc_body)


def kernel(positions_flat, obj_id, tables):
    xs = positions_flat[:, 0]
    ys = positions_flat[:, 1]
    zs = positions_flat[:, 2]
    tab = lax.dynamic_index_in_dim(tables, jnp.asarray(obj_id), 0,
                                   keepdims=False)
    tb = lax.bitcast_convert_type(tab.astype(jnp.bfloat16), jnp.uint16)
    packed = tb[:, 0].astype(jnp.uint32) | (tb[:, 1].astype(jnp.uint32) << 16)
    tabp = lax.bitcast_convert_type(packed, jnp.float32)
    parts = []
    for l in range(_NUM_LEVELS):
        parts.append(tabp[_OFFSETS[l]:_OFFSETS[l] + _SIZES[l]])
        nxt = _AOFF[l + 1] if l + 1 < _NUM_LEVELS else _RPAD
        gap = nxt - (_AOFF[l] + _SIZES[l])
        if gap:
            parts.append(jnp.zeros((gap,), jnp.float32))
    tabp = jnp.concatenate(parts)
    out = _hashgrid_sc(xs, ys, zs, tabp)
    return out.T


# speedup vs baseline: 1.2238x; 1.2238x over previous
import functools

import jax
import jax.numpy as jnp
import numpy as np
from jax import lax
from jax.experimental import pallas as pl
from jax.experimental.pallas import tpu as pltpu
from jax.experimental.pallas import tpu_sc as plsc

_NUM_OBJ = 4
_NUM_LEVELS = 16
_FPL = 2
_T = 1 << 19
_BASE_RES = 16
_GROWTH = 1.3819
_N = 524288

_PY = np.int32(np.uint32(2654435761).view(np.int32))
_PZ = np.int32(805459861)


def _levels():
    ress, sizes, offsets = [], [], []
    off = 0
    for l in range(_NUM_LEVELS):
        res = int(np.floor(_BASE_RES * (_GROWTH ** l)))
        nv = (res + 1) ** 3
        size = min(nv, _T)
        ress.append(res)
        sizes.append(size)
        offsets.append(off)
        off += size
    return ress, sizes, offsets, off


_RESS, _SIZES, _OFFSETS, _TOTAL_ROWS = _levels()
_N_DENSE = sum(1 for l in range(_NUM_LEVELS)
               if _SIZES[l] == (_RESS[l] + 1) ** 3)

_PIECE = 8192
_DPIECES = -(-_OFFSETS[_N_DENSE] // _PIECE)
_SWS = (_T // _PIECE + 1) * _PIECE
_SAL = [(off // 8) * 8 for off in _OFFSETS]
_REL = [off - sal for off, sal in zip(_OFFSETS, _SAL)]
_RPAD = _SAL[_NUM_LEVELS - 1] + _SWS
_NC, _NS = 2, 16
_NW = _NC * _NS
_C = 256
_PPW = _N // _NW
_NCH = _PPW // _C
_HALF = _NCH // 2

_CORNERS = [(dx, dy, dz) for dx in (0, 1) for dy in (0, 1) for dz in (0, 1)]


def _sc_body(xs_hbm, ys_hbm, zs_hbm, tab_hbm, out_hbm,
             px, py, pz, idx_a, idx_b, rows_a, rows_b, wt_a, wt_b, out_h,
             bounce_a, bounce_b, spm, sem_a, sem_b):
    cid = lax.axis_index("c")
    sid = lax.axis_index("s")
    wid = cid * _NS + sid

    pstart = pl.multiple_of(wid * _PPW, 8)
    pltpu.sync_copy(xs_hbm.at[pl.ds(pstart, _PPW)], px)
    pltpu.sync_copy(ys_hbm.at[pl.ds(pstart, _PPW)], py)
    pltpu.sync_copy(zs_hbm.at[pl.ds(pstart, _PPW)], pz)

    def stage(start, npieces):
        bufs, sems = [bounce_a, bounce_b], [sem_a, sem_b]

        def fetch(k):
            p = k * _NS + sid

            @pl.when(p < npieces)
            def _():
                src = pl.multiple_of(start + p * _PIECE, 8)
                pltpu.async_copy(tab_hbm.at[pl.ds(src, _PIECE)],
                                 bufs[k % 2], sems[k % 2])

        def push(k):
            p = k * _NS + sid

            @pl.when(p < npieces)
            def _():
                pltpu.make_async_copy(tab_hbm.at[pl.ds(0, _PIECE)],
                                      bufs[k % 2], sems[k % 2]).wait()
                dst = pl.multiple_of(p * _PIECE, 8)
                pltpu.sync_copy(bufs[k % 2], spm.at[pl.ds(dst, _PIECE)])

        nk = -(-npieces // _NS)
        fetch(0)
        for k in range(1, nk):
            fetch(k)
            push(k - 1)
        push(nk - 1)

    stage(0, _DPIECES)
    plsc.subcore_barrier()

    def compute_idx(l, ck, idx_buf, wt_buf):
        res, size = _RESS[l], _SIZES[l]
        dense = size == (res + 1) ** 3
        rel = _OFFSETS[l] if dense else _REL[l]
        fr = jnp.float32(res)

        def g_body(g, c_):
            s = g * 16
            o = ck * _C + s
            x = px[pl.ds(o, 16)]
            y = py[pl.ds(o, 16)]
            z = pz[pl.ds(o, 16)]
            xf, yf, zf = x * fr, y * fr, z * fr
            xi = xf.astype(jnp.int32)
            yi = yf.astype(jnp.int32)
            zi = zf.astype(jnp.int32)
            wx = xf - xi.astype(jnp.float32)
            wy = yf - yi.astype(jnp.float32)
            wz = zf - zi.astype(jnp.float32)
            cwx, cwy, cwz = 1.0 - wx, 1.0 - wy, 1.0 - wz
            wyz = {(0, 0): cwy * cwz, (0, 1): cwy * wz,
                   (1, 0): wy * cwz, (1, 1): wy * wz}
            for c, (dx, dy, dz) in enumerate(_CORNERS):
                wt_buf[pl.ds(c * _C + s, 16)] = \
                    (wx if dx else cwx) * wyz[(dy, dz)]
            cx = jnp.minimum(xi, res - 1)
            cy = jnp.minimum(yi, res - 1)
            cz = jnp.minimum(zi, res - 1)
            if dense:
                r1 = res + 1
                a00 = cy + r1 * cz
                ra = {(0, 0): r1 * a00 + rel,
                      (0, 1): r1 * a00 + r1 * r1 + rel,
                      (1, 0): r1 * a00 + r1 + rel,
                      (1, 1): r1 * a00 + r1 * r1 + r1 + rel}
                cx1 = cx + 1
                for c, (dx, dy, dz) in enumerate(_CORNERS):
                    idx_buf[pl.ds(c * _C + s, 16)] = \
                        (cx1 if dx else cx) + ra[(dy, dz)]
            else:
                mask = size - 1
                hx0, hx1 = cx, cx + 1
                hy0 = cy * _PY
                hy1 = hy0 + _PY
                hz0 = cz * _PZ
                hz1 = hz0 + _PZ
                e = {(0, 0): hy0 ^ hz0, (0, 1): hy0 ^ hz1,
                     (1, 0): hy1 ^ hz0, (1, 1): hy1 ^ hz1}
                for c, (dx, dy, dz) in enumerate(_CORNERS):
                    h = (hx1 if dx else hx0) ^ e[(dy, dz)]
                    idx_buf[pl.ds(c * _C + s, 16)] = (h & mask) + rel
            return c_

        lax.fori_loop(0, _C // 16, g_body, 0)

    def accumulate(l, ck, rows_buf, wt_buf):
        himask = jnp.int32(-65536)

        def g_body(g, c_):
            s = g * 16
            acc0 = jnp.zeros((16,), jnp.float32)
            acc1 = jnp.zeros((16,), jnp.float32)
            for c, (dx, dy, dz) in enumerate(_CORNERS):
                wt = wt_buf[pl.ds(c * _C + s, 16)]
                w = lax.bitcast_convert_type(rows_buf[pl.ds(c * _C + s, 16)],
                                             jnp.int32)
                f0 = lax.bitcast_convert_type(lax.shift_left(w, 16),
                                              jnp.float32)
                f1 = lax.bitcast_convert_type(w & himask, jnp.float32)
                acc0 = acc0 + wt * f0
                acc1 = acc1 + wt * f1
            oo = (ck & (_HALF - 1)) * _C + s
            out_h[0, pl.ds(oo, 16)] = acc0
            out_h[1, pl.ds(oo, 16)] = acc1
            return c_

        lax.fori_loop(0, _C // 16, g_body, 0)

    def flush(l, ck):
        @pl.when((ck & (_HALF - 1)) == _HALF - 1)
        def _():
            off = pl.multiple_of(wid * _PPW + (ck // _HALF) * (_HALF * _C), 8)
            pltpu.sync_copy(out_h.at[0],
                            out_hbm.at[2 * l].at[pl.ds(off, _HALF * _C)])
            pltpu.sync_copy(out_h.at[1],
                            out_hbm.at[2 * l + 1].at[pl.ds(off, _HALF * _C)])

    for l in range(_NUM_LEVELS):
        if l >= _N_DENSE:
            plsc.subcore_barrier()
            stage(_SAL[l], _SWS // _PIECE)
            plsc.subcore_barrier()
        src = spm

        compute_idx(l, 0, idx_a, wt_a)
        pltpu.async_copy(src.at[idx_a], rows_a, sem_a)

        def pair_body(j, c_, l=l, src=src):
            ck_b = 2 * j + 1
            compute_idx(l, ck_b, idx_b, wt_b)
            pltpu.async_copy(src.at[idx_b], rows_b, sem_b)
            pltpu.make_async_copy(src.at[idx_a], rows_a, sem_a).wait()
            accumulate(l, 2 * j, rows_a, wt_a)
            ck_a = jnp.minimum(2 * j + 2, _NCH - 1)
            compute_idx(l, ck_a, idx_a, wt_a)
            pltpu.async_copy(src.at[idx_a], rows_a, sem_a)
            pltpu.make_async_copy(src.at[idx_b], rows_b, sem_b).wait()
            accumulate(l, ck_b, rows_b, wt_b)
            flush(l, ck_b)
            return c_

        lax.fori_loop(0, _NCH // 2, pair_body, 0)
        pltpu.make_async_copy(src.at[idx_a], rows_a, sem_a).wait()


_hashgrid_sc = functools.partial(
    pl.kernel,
    out_type=jax.ShapeDtypeStruct((2 * _NUM_LEVELS, _N), jnp.float32),
    mesh=plsc.VectorSubcoreMesh(core_axis_name="c", subcore_axis_name="s",
                                num_cores=_NC, num_subcores=_NS),
    scratch_types=[
        pltpu.VMEM((_PPW,), jnp.float32),
        pltpu.VMEM((_PPW,), jnp.float32),
        pltpu.VMEM((_PPW,), jnp.float32),
        pltpu.VMEM((8 * _C,), jnp.int32),
        pltpu.VMEM((8 * _C,), jnp.int32),
        pltpu.VMEM((8 * _C,), jnp.float32),
        pltpu.VMEM((8 * _C,), jnp.float32),
        pltpu.VMEM((8 * _C,), jnp.float32),
        pltpu.VMEM((8 * _C,), jnp.float32),
        pltpu.VMEM((2, _HALF * _C), jnp.float32),
        pltpu.VMEM((_PIECE,), jnp.float32),
        pltpu.VMEM((_PIECE,), jnp.float32),
        pltpu.VMEM_SHARED((_SWS,), jnp.float32),
        pltpu.SemaphoreType.DMA,
        pltpu.SemaphoreType.DMA,
    ],
)(_sc_body)


def kernel(positions_flat, obj_id, tables):
    xs = positions_flat[:, 0]
    ys = positions_flat[:, 1]
    zs = positions_flat[:, 2]
    tab = lax.dynamic_index_in_dim(tables, jnp.asarray(obj_id), 0,
                                   keepdims=False)
    tb = lax.bitcast_convert_type(tab.astype(jnp.bfloat16), jnp.uint16)
    packed = tb[:, 0].astype(jnp.uint32) | (tb[:, 1].astype(jnp.uint32) << 16)
    tabp = lax.bitcast_convert_type(packed, jnp.float32)
    tabp = jnp.concatenate(
        [tabp, jnp.zeros((_RPAD - _TOTAL_ROWS,), jnp.float32)])
    out = _hashgrid_sc(xs, ys, zs, tabp)
    return out.T
